# Initial kernel scaffold; baseline (speedup 1.0000x reference)
import jax
import jax.numpy as jnp
from jax.experimental import pallas as pl


def kernel(indices, emb_table):
    raise NotImplementedError("write your pallas kernel here")


# speedup vs baseline: 1.6824x; 1.6824x over previous
import functools

import jax
import jax.numpy as jnp
from jax import lax
from jax.experimental import pallas as pl
from jax.experimental.pallas import tpu as pltpu
from jax.experimental.pallas import tpu_sc as plsc

_CHUNK = 128


def kernel(indices, emb_table):
    bsz, hist = indices.shape
    vocab, dim = emb_table.shape
    n = bsz * hist

    mesh = plsc.VectorSubcoreMesh(core_axis_name="c", subcore_axis_name="s")
    nc, ns = mesh.num_cores, mesh.num_subcores
    nw = nc * ns
    n_chunks = n // (nw * _CHUNK)
    assert n == nw * n_chunks * _CHUNK, (n, nw, n_chunks)

    idx3 = indices.reshape(nw, n_chunks, _CHUNK).astype(jnp.int32)

    @functools.partial(
        pl.kernel,
        out_type=jax.ShapeDtypeStruct((nw, n_chunks, _CHUNK, dim), jnp.float32),
        mesh=mesh,
        scratch_types=[
            pltpu.VMEM((n_chunks, _CHUNK), jnp.int32),
            pltpu.VMEM((_CHUNK, dim), jnp.float32),
            pltpu.SemaphoreType.DMA,
        ],
        compiler_params=pltpu.CompilerParams(use_tc_tiling_on_sc=False),
    )
    def emb_gather(table_hbm, idx_hbm, out_hbm, idx_v, rows_v, sem):
        wid = lax.axis_index("s") * nc + lax.axis_index("c")
        pltpu.sync_copy(idx_hbm.at[wid], idx_v)

        @pl.loop(0, n_chunks)
        def _chunk(c):
            pltpu.async_copy(table_hbm.at[idx_v.at[c]], rows_v, sem).wait()
            pltpu.sync_copy(rows_v, out_hbm.at[wid, c])

    out = emb_gather(emb_table, idx3)
    return out.reshape(bsz, hist, dim)


# speedup vs baseline: 1.8747x; 1.1143x over previous
import functools

import jax
import jax.numpy as jnp
from jax import lax
from jax.experimental import pallas as pl
from jax.experimental.pallas import tpu as pltpu
from jax.experimental.pallas import tpu_sc as plsc

_CHUNK = 128
_K = 4


def kernel(indices, emb_table):
    bsz, hist = indices.shape
    vocab, dim = emb_table.shape
    n = bsz * hist

    mesh = plsc.VectorSubcoreMesh(core_axis_name="c", subcore_axis_name="s")
    nc, ns = mesh.num_cores, mesh.num_subcores
    nw = nc * ns
    n_chunks = n // (nw * _CHUNK)
    assert n == nw * n_chunks * _CHUNK, (n, nw, n_chunks)
    n_groups = n_chunks // _K
    assert n_chunks == n_groups * _K and n_groups % 2 == 0, (n_chunks, n_groups)
    rows_per_w = n_chunks * _CHUNK
    grp_rows = _K * _CHUNK

    idx3 = indices.reshape(nw, n_chunks, _CHUNK).astype(jnp.int32)

    @functools.partial(
        pl.kernel,
        out_type=jax.ShapeDtypeStruct((nw, rows_per_w, dim), jnp.float32),
        mesh=mesh,
        scratch_types=[
            pltpu.VMEM((n_chunks, _CHUNK), jnp.int32),
            pltpu.VMEM((2, grp_rows, dim), jnp.float32),
            pltpu.SemaphoreType.DMA,
            pltpu.SemaphoreType.DMA,
            pltpu.SemaphoreType.DMA,
            pltpu.SemaphoreType.DMA,
        ],
        compiler_params=pltpu.CompilerParams(use_tc_tiling_on_sc=False),
    )
    def emb_gather(table_hbm, idx_hbm, out_hbm, idx_v, rows_v, sg0, sg1, ss0, ss1):
        wid = lax.axis_index("s") * nc + lax.axis_index("c")
        sem_g = (sg0, sg1)
        sem_s = (ss0, ss1)
        pltpu.sync_copy(idx_hbm.at[wid], idx_v)

        def fire_gathers(grp, half):
            for b in range(_K):
                c = grp * _K + b
                pltpu.async_copy(
                    table_hbm.at[idx_v.at[c]],
                    rows_v.at[half, pl.ds(b * _CHUNK, _CHUNK)],
                    sem_g[half],
                )

        def drain_gathers(grp, half):
            for b in range(_K):
                c = grp * _K + b
                pltpu.make_async_copy(
                    table_hbm.at[idx_v.at[c]],
                    rows_v.at[half, pl.ds(b * _CHUNK, _CHUNK)],
                    sem_g[half],
                ).wait()

        def start_store(grp, half):
            pltpu.async_copy(
                rows_v.at[half],
                out_hbm.at[wid, pl.ds(grp * grp_rows, grp_rows)],
                sem_s[half],
            )

        def wait_store(grp, half):
            pltpu.make_async_copy(
                rows_v.at[half],
                out_hbm.at[wid, pl.ds(grp * grp_rows, grp_rows)],
                sem_s[half],
            ).wait()

        fire_gathers(0, 0)

        @pl.loop(0, n_groups, step=2)
        def _grp(g):
            for sel in (0, 1):
                gg = g + sel
                nxt = gg + 1
                other = 1 - sel

                @pl.when(nxt < n_groups)
                def _fire_next():
                    @pl.when(nxt >= 2)
                    def _wait_prev_store():
                        wait_store(nxt - 2, other)

                    fire_gathers(nxt, other)

                drain_gathers(gg, sel)
                start_store(gg, sel)

        wait_store(n_groups - 2, 0)
        wait_store(n_groups - 1, 1)

    out = emb_gather(emb_table, idx3)
    return out.reshape(bsz, hist, dim)
